# Initial kernel scaffold; baseline (speedup 1.0000x reference)
import jax
import jax.numpy as jnp
from jax.experimental import pallas as pl


def kernel(x, w1, b1, w2, b2, prototypes):
    raise NotImplementedError("write your pallas kernel here")


# speedup vs baseline: 1.3012x; 1.3012x over previous
import jax
import jax.numpy as jnp
from jax.experimental import pallas as pl
from jax.experimental.pallas import tpu as pltpu

B, D, H = 4096, 1024, 2048
BM = 256
OP = 16


def _dense_body(x_ref, w1_ref, b1_ref, w2_ref, b2_ref, p_ref, o_ref):
    xb = x_ref[...]
    p = p_ref[...]
    diff0 = xb - p[0:1, :]
    diff1 = xb - p[1:2, :]
    d0 = jnp.sqrt(jnp.sum(diff0 * diff0, axis=1, keepdims=True))
    d1 = jnp.sqrt(jnp.sum(diff1 * diff1, axis=1, keepdims=True))
    pick1 = d1 < d0

    dn = (((1,), (1,)), ((), ()))
    h0 = jax.nn.relu(
        jax.lax.dot_general(xb, w1_ref[0], dn, preferred_element_type=jnp.float32)
        + b1_ref[0][None, :])
    o0 = (jax.lax.dot_general(h0, w2_ref[0], dn, preferred_element_type=jnp.float32)
          + b2_ref[0][None, :])
    h1 = jax.nn.relu(
        jax.lax.dot_general(xb, w1_ref[1], dn, preferred_element_type=jnp.float32)
        + b1_ref[1][None, :])
    o1 = (jax.lax.dot_general(h1, w2_ref[1], dn, preferred_element_type=jnp.float32)
          + b2_ref[1][None, :])
    o_ref[...] = jnp.where(pick1, o1, o0)


def kernel(x, w1, b1, w2, b2, prototypes):
    w2p = jnp.zeros((2, OP, H), jnp.float32).at[:, :2, :].set(w2)
    b2p = jnp.zeros((2, OP), jnp.float32).at[:, :2].set(b2)

    out = pl.pallas_call(
        _dense_body,
        grid=(B // BM,),
        in_specs=[
            pl.BlockSpec((BM, D), lambda i: (i, 0)),
            pl.BlockSpec((2, H, D), lambda i: (0, 0, 0)),
            pl.BlockSpec((2, H), lambda i: (0, 0)),
            pl.BlockSpec((2, OP, H), lambda i: (0, 0, 0)),
            pl.BlockSpec((2, OP), lambda i: (0, 0)),
            pl.BlockSpec((2, D), lambda i: (0, 0)),
        ],
        out_specs=pl.BlockSpec((BM, OP), lambda i: (i, 0)),
        out_shape=jax.ShapeDtypeStruct((B, OP), jnp.float32),
    )(x, w1, b1, w2p, b2p, prototypes)
    return out[:, :2]
